# Initial kernel scaffold; baseline (speedup 1.0000x reference)
import jax
import jax.numpy as jnp
from jax.experimental import pallas as pl


def kernel(z, embedding_weight):
    raise NotImplementedError("write your pallas kernel here")


# speedup vs baseline: 1.2437x; 1.2437x over previous
import functools

import jax
import jax.numpy as jnp
from jax import lax
from jax.experimental import pallas as pl
from jax.experimental.pallas import tpu as pltpu
from jax.experimental.pallas import tpu_sc as plsc

_N_E = 8192
_DIM = 256
_BM = 256
_BETA = 0.25


def _dist_argmin_body(z_ref, et_ref, idx_ref, mind_ref, b_ref):
    @pl.when(pl.program_id(0) == 0)
    def _():
        et0 = et_ref[...]
        b_ref[...] = jnp.sum(et0 * et0, axis=0, keepdims=True)

    z = z_ref[...]
    a = jnp.sum(z * z, axis=1, keepdims=True)
    s = jnp.dot(z, et_ref[...], preferred_element_type=jnp.float32)
    d = (a + b_ref[...]) - 2.0 * s
    m = jnp.min(d, axis=1, keepdims=True)
    ii = lax.broadcasted_iota(jnp.int32, d.shape, 1)
    idx = jnp.min(jnp.where(d == m, ii, jnp.int32(_N_E)), axis=1)
    idx_ref[0, 0, :] = idx
    mind_ref[0, 0, :] = m[:, 0]


_GRID = _N_E // _BM

_dist_argmin = pl.pallas_call(
    _dist_argmin_body,
    grid=(_GRID,),
    in_specs=[
        pl.BlockSpec((_BM, _DIM), lambda i: (i, 0)),
        pl.BlockSpec((_DIM, _N_E), lambda i: (0, 0)),
    ],
    out_specs=[
        pl.BlockSpec((1, 1, _BM), lambda i: (i, 0, 0)),
        pl.BlockSpec((1, 1, _BM), lambda i: (i, 0, 0)),
    ],
    out_shape=[
        jax.ShapeDtypeStruct((_GRID, 1, _BM), jnp.int32),
        jax.ShapeDtypeStruct((_GRID, 1, _BM), jnp.float32),
    ],
    scratch_shapes=[pltpu.VMEM((1, _N_E), jnp.float32)],
)


def _make_sc_gather(n_rows, dim):
    info = plsc.get_sparse_core_info()
    nc, ns = info.num_cores, info.num_subcores
    nw = nc * ns
    rows_per_w = n_rows // nw
    mesh = plsc.VectorSubcoreMesh(core_axis_name="c", subcore_axis_name="s")

    @functools.partial(
        pl.kernel,
        mesh=mesh,
        out_type=jax.ShapeDtypeStruct((n_rows, dim), jnp.float32),
        scratch_types=[
            pltpu.VMEM((rows_per_w,), jnp.int32),
            pltpu.VMEM((rows_per_w, dim), jnp.float32),
            pltpu.SemaphoreType.DMA,
        ],
    )
    def _gather(table_hbm, idx_hbm, out_hbm, idx_v, rows_v, sem):
        wid = lax.axis_index("s") * nc + lax.axis_index("c")
        base = wid * rows_per_w
        pltpu.sync_copy(idx_hbm.at[pl.ds(base, rows_per_w)], idx_v)
        pltpu.async_copy(table_hbm.at[idx_v], rows_v, sem).wait()
        pltpu.sync_copy(rows_v, out_hbm.at[pl.ds(base, rows_per_w)])

    return _gather


def kernel(z, embedding_weight):
    b, c, h, w = z.shape
    z_flat = jnp.transpose(z, (0, 2, 3, 1)).reshape(-1, c)
    et = embedding_weight.T
    idx3, mind3 = _dist_argmin(z_flat, et)
    idx = idx3.reshape(-1)
    gather = _make_sc_gather(z_flat.shape[0], c)
    z_q_flat = gather(embedding_weight, idx)
    z_q = jnp.transpose(z_q_flat.reshape(b, h, w, c), (0, 3, 1, 2))
    loss = (1.0 + _BETA) * jnp.sum(mind3) / z.size
    return (z_q, loss)
